# initial kernel scaffold (unmeasured)
import jax
import jax.numpy as jnp
from jax import lax
from jax.experimental import pallas as pl
from jax.experimental.pallas import tpu as pltpu


def kernel(partial, gamma):
    _, m, d = partial.shape
    half = m // 2

    def body(p_ref, g_ref, out_ref, comm_ref, send_sem, recv_sem):
        my_x = lax.axis_index("x")
        my_y = lax.axis_index("y")
        peer_y = 1 - my_y

        barrier_sem = pltpu.get_barrier_semaphore()
        pl.semaphore_signal(
            barrier_sem,
            inc=1,
            device_id=(my_x, peer_y),
            device_id_type=pl.DeviceIdType.MESH,
        )
        pl.semaphore_wait(barrier_sem, 1)

        rdma = pltpu.make_async_remote_copy(
            src_ref=p_ref.at[0, pl.ds(peer_y * half, half), :],
            dst_ref=comm_ref,
            send_sem=send_sem,
            recv_sem=recv_sem,
            device_id=(my_x, peer_y),
            device_id_type=pl.DeviceIdType.MESH,
        )
        rdma.start()
        rdma.wait()

        y = p_ref[0, pl.ds(my_y * half, half), :] + comm_ref[...]
        ms = jnp.mean(y * y, axis=-1, keepdims=True)
        out_ref[...] = (y * lax.rsqrt(ms + 1e-6)) * g_ref[...]

    return pl.pallas_call(
        body,
        out_shape=jax.ShapeDtypeStruct((half, d), jnp.float32),
        in_specs=[
            pl.BlockSpec(memory_space=pltpu.VMEM),
            pl.BlockSpec(memory_space=pltpu.VMEM),
        ],
        out_specs=pl.BlockSpec(memory_space=pltpu.VMEM),
        scratch_shapes=[
            pltpu.VMEM((half, d), jnp.float32),
            pltpu.SemaphoreType.DMA,
            pltpu.SemaphoreType.DMA,
        ],
        compiler_params=pltpu.CompilerParams(collective_id=0),
    )(partial, gamma.reshape(1, d))


# baseline (device time: 207728 ns/iter reference)
import jax
import jax.numpy as jnp
from jax import lax
from jax.experimental import pallas as pl
from jax.experimental.pallas import tpu as pltpu

CHUNK = 512


def kernel(partial, gamma):
    _, m, d = partial.shape
    half = m // 2
    n_chunks = half // CHUNK

    def body(
        p_ref,
        g_ref,
        out_ref,
        comm_ref,
        loc_ref,
        rcv_ref,
        send_sem,
        recv_sem,
        loc_sem,
        rcv_sem,
    ):
        my_x = lax.axis_index("x")
        my_y = lax.axis_index("y")
        peer_y = 1 - my_y

        barrier_sem = pltpu.get_barrier_semaphore()
        pl.semaphore_signal(
            barrier_sem,
            inc=1,
            device_id=(my_x, peer_y),
            device_id_type=pl.DeviceIdType.MESH,
        )
        pl.semaphore_wait(barrier_sem, 1)

        rdma = pltpu.make_async_remote_copy(
            src_ref=p_ref.at[0, pl.ds(peer_y * half, half), :],
            dst_ref=comm_ref,
            send_sem=send_sem,
            recv_sem=recv_sem,
            device_id=(my_x, peer_y),
            device_id_type=pl.DeviceIdType.MESH,
        )
        rdma.start()
        rdma.wait()

        for c in range(n_chunks):
            cp_loc = pltpu.make_async_copy(
                p_ref.at[0, pl.ds(my_y * half + c * CHUNK, CHUNK), :],
                loc_ref,
                loc_sem,
            )
            cp_rcv = pltpu.make_async_copy(
                comm_ref.at[pl.ds(c * CHUNK, CHUNK), :],
                rcv_ref,
                rcv_sem,
            )
            cp_loc.start()
            cp_rcv.start()
            cp_loc.wait()
            cp_rcv.wait()

            y = loc_ref[...] + rcv_ref[...]
            ms = jnp.mean(y * y, axis=-1, keepdims=True)
            out_ref[pl.ds(c * CHUNK, CHUNK), :] = (
                y * lax.rsqrt(ms + 1e-6)
            ) * g_ref[...]

    out, _comm = pl.pallas_call(
        body,
        out_shape=[
            jax.ShapeDtypeStruct((half, d), jnp.float32),
            jax.ShapeDtypeStruct((half, d), jnp.float32),
        ],
        in_specs=[
            pl.BlockSpec(memory_space=pl.ANY),
            pl.BlockSpec(memory_space=pltpu.VMEM),
        ],
        out_specs=[
            pl.BlockSpec(memory_space=pltpu.VMEM),
            pl.BlockSpec(memory_space=pl.ANY),
        ],
        scratch_shapes=[
            pltpu.VMEM((CHUNK, d), jnp.float32),
            pltpu.VMEM((CHUNK, d), jnp.float32),
            pltpu.SemaphoreType.DMA,
            pltpu.SemaphoreType.DMA,
            pltpu.SemaphoreType.DMA,
            pltpu.SemaphoreType.DMA,
        ],
        compiler_params=pltpu.CompilerParams(collective_id=0),
    )(partial, gamma.reshape(1, d))
    return out


# device time: 109898 ns/iter; 1.8902x vs baseline; 1.8902x over previous
import jax
import jax.numpy as jnp
from jax import lax
from jax.experimental import pallas as pl
from jax.experimental.pallas import tpu as pltpu

R = 64


def kernel(partial, gamma):
    _, m, d = partial.shape
    half = m // 2
    quarter = half // 2
    n = quarter // R

    def body(
        p_ref,
        g_ref,
        out_ref,
        ybuf,
        locbuf,
        ysend,
        yrecv,
        xsend,
        xrecv,
        loc_sem,
    ):
        my_x = lax.axis_index("x")
        my_y = lax.axis_index("y")
        peer_y_dev = (my_x, 1 - my_y)
        peer_x_dev = (1 - my_x, my_y)

        barrier_sem = pltpu.get_barrier_semaphore()
        for dev in (peer_y_dev, peer_x_dev):
            pl.semaphore_signal(
                barrier_sem,
                inc=1,
                device_id=dev,
                device_id_type=pl.DeviceIdType.MESH,
            )
        pl.semaphore_wait(barrier_sem, 2)

        qstart = my_x * quarter
        loc_base = my_y * half + qstart
        ysrc_base = (1 - my_y) * half + qstart

        cp_loc = pltpu.make_async_copy(
            p_ref.at[0, pl.ds(loc_base, quarter), :], locbuf, loc_sem
        )
        cp_loc.start()

        y_rdmas = []
        for c in range(n):
            r = pltpu.make_async_remote_copy(
                src_ref=p_ref.at[0, pl.ds(ysrc_base + c * R, R), :],
                dst_ref=ybuf.at[pl.ds(c * R, R), :],
                send_sem=ysend.at[c],
                recv_sem=yrecv.at[c],
                device_id=peer_y_dev,
                device_id_type=pl.DeviceIdType.MESH,
            )
            r.start()
            y_rdmas.append(r)

        cp_loc.wait()

        x_rdmas = []
        for c in range(n):
            y_rdmas[c].wait_recv()
            s = locbuf[c * R : (c + 1) * R, :] + ybuf[c * R : (c + 1) * R, :]
            ms = jnp.mean(s * s, axis=-1, keepdims=True)
            out_ref[pl.ds(qstart + c * R, R), :] = (
                s * lax.rsqrt(ms + 1e-6)
            ) * g_ref[...]
            r = pltpu.make_async_remote_copy(
                src_ref=out_ref.at[pl.ds(qstart + c * R, R), :],
                dst_ref=out_ref.at[pl.ds(qstart + c * R, R), :],
                send_sem=xsend.at[c],
                recv_sem=xrecv.at[c],
                device_id=peer_x_dev,
                device_id_type=pl.DeviceIdType.MESH,
            )
            r.start()
            x_rdmas.append(r)

        for c in range(n):
            x_rdmas[c].wait_recv()
            x_rdmas[c].wait_send()
            y_rdmas[c].wait_send()

    return pl.pallas_call(
        body,
        out_shape=jax.ShapeDtypeStruct((half, d), jnp.float32),
        in_specs=[
            pl.BlockSpec(memory_space=pl.ANY),
            pl.BlockSpec(memory_space=pltpu.VMEM),
        ],
        out_specs=pl.BlockSpec(memory_space=pltpu.VMEM),
        scratch_shapes=[
            pltpu.VMEM((quarter, d), jnp.float32),
            pltpu.VMEM((quarter, d), jnp.float32),
            pltpu.SemaphoreType.DMA((n,)),
            pltpu.SemaphoreType.DMA((n,)),
            pltpu.SemaphoreType.DMA((n,)),
            pltpu.SemaphoreType.DMA((n,)),
            pltpu.SemaphoreType.DMA,
        ],
        compiler_params=pltpu.CompilerParams(collective_id=0),
    )(partial, gamma.reshape(1, d))


# device time: 107789 ns/iter; 1.9272x vs baseline; 1.0196x over previous
import jax
import jax.numpy as jnp
from jax import lax
from jax.experimental import pallas as pl
from jax.experimental.pallas import tpu as pltpu

R = 32


def kernel(partial, gamma):
    _, m, d = partial.shape
    half = m // 2
    quarter = half // 2
    n = quarter // R

    def body(
        p_ref,
        g_ref,
        out_ref,
        ybuf,
        locbuf,
        ysend,
        yrecv,
        xsend,
        xrecv,
        loc_sem,
    ):
        my_x = lax.axis_index("x")
        my_y = lax.axis_index("y")
        peer_y_dev = (my_x, 1 - my_y)
        peer_x_dev = (1 - my_x, my_y)

        qstart = my_x * quarter
        loc_base = my_y * half + qstart
        ysrc_base = (1 - my_y) * half + qstart

        loc_copies = []
        for c in range(n):
            cp = pltpu.make_async_copy(
                p_ref.at[0, pl.ds(loc_base + c * R, R), :],
                locbuf.at[pl.ds(c * R, R), :],
                loc_sem.at[c],
            )
            cp.start()
            loc_copies.append(cp)

        barrier_sem = pltpu.get_barrier_semaphore()
        for dev in (peer_y_dev, peer_x_dev):
            pl.semaphore_signal(
                barrier_sem,
                inc=1,
                device_id=dev,
                device_id_type=pl.DeviceIdType.MESH,
            )
        pl.semaphore_wait(barrier_sem, 2)

        y_rdmas = []
        for c in range(n):
            r = pltpu.make_async_remote_copy(
                src_ref=p_ref.at[0, pl.ds(ysrc_base + c * R, R), :],
                dst_ref=ybuf.at[pl.ds(c * R, R), :],
                send_sem=ysend.at[c],
                recv_sem=yrecv.at[c],
                device_id=peer_y_dev,
                device_id_type=pl.DeviceIdType.MESH,
            )
            r.start()
            y_rdmas.append(r)

        x_rdmas = []
        for c in range(n):
            loc_copies[c].wait()
            y_rdmas[c].wait_recv()
            s = locbuf[c * R : (c + 1) * R, :] + ybuf[c * R : (c + 1) * R, :]
            ms = jnp.mean(s * s, axis=-1, keepdims=True)
            out_ref[pl.ds(qstart + c * R, R), :] = (
                s * lax.rsqrt(ms + 1e-6)
            ) * g_ref[...]
            r = pltpu.make_async_remote_copy(
                src_ref=out_ref.at[pl.ds(qstart + c * R, R), :],
                dst_ref=out_ref.at[pl.ds(qstart + c * R, R), :],
                send_sem=xsend.at[c],
                recv_sem=xrecv.at[c],
                device_id=peer_x_dev,
                device_id_type=pl.DeviceIdType.MESH,
            )
            r.start()
            x_rdmas.append(r)

        for c in range(n):
            x_rdmas[c].wait_recv()
            x_rdmas[c].wait_send()
            y_rdmas[c].wait_send()

    return pl.pallas_call(
        body,
        out_shape=jax.ShapeDtypeStruct((half, d), jnp.float32),
        in_specs=[
            pl.BlockSpec(memory_space=pl.ANY),
            pl.BlockSpec(memory_space=pltpu.VMEM),
        ],
        out_specs=pl.BlockSpec(memory_space=pltpu.VMEM),
        scratch_shapes=[
            pltpu.VMEM((quarter, d), jnp.float32),
            pltpu.VMEM((quarter, d), jnp.float32),
            pltpu.SemaphoreType.DMA((n,)),
            pltpu.SemaphoreType.DMA((n,)),
            pltpu.SemaphoreType.DMA((n,)),
            pltpu.SemaphoreType.DMA((n,)),
            pltpu.SemaphoreType.DMA((n,)),
        ],
        compiler_params=pltpu.CompilerParams(collective_id=0),
    )(partial, gamma.reshape(1, d))
